# baseline (speedup 1.0000x reference)
import functools

import jax
import jax.numpy as jnp
from jax import lax
from jax.experimental import pallas as pl
from jax.experimental.pallas import tpu as pltpu
from jax.experimental.pallas import tpu_sc as plsc

_NUM_BAGS = 16
_NUM_INST = 256
_G = 4096
_VOCAB = 128
_NC = 2
_NS = 16
_NW = _NC * _NS
_GPW = _G // _NW
_L = 16


@functools.cache
def _make_ig_gather():
    mesh = plsc.VectorSubcoreMesh(core_axis_name="c", subcore_axis_name="s")
    half = _GPW // 2

    @functools.partial(
        pl.kernel,
        mesh=mesh,
        out_type=jax.ShapeDtypeStruct((_G, _L), jnp.float32),
        scratch_types=[
            pltpu.VMEM((_GPW,), jnp.int32),
            pltpu.VMEM((_GPW, _L), jnp.float32),
            pltpu.SemaphoreType.DMA,
            pltpu.SemaphoreType.DMA,
        ],
        compiler_params=pltpu.CompilerParams(use_tc_tiling_on_sc=False),
    )
    def ig_gather(table_hbm, idx_hbm, out_hbm, idx_v, rows_v, sem0, sem1):
        wid = lax.axis_index("s") * _NC + lax.axis_index("c")
        base = wid * _GPW
        pltpu.sync_copy(idx_hbm.at[pl.ds(base, _GPW)], idx_v)
        cp0 = pltpu.async_copy(
            table_hbm.at[idx_v.at[pl.ds(0, half)]],
            rows_v.at[pl.ds(0, half)], sem0)
        cp1 = pltpu.async_copy(
            table_hbm.at[idx_v.at[pl.ds(half, half)]],
            rows_v.at[pl.ds(half, half)], sem1)
        lane = lax.iota(jnp.int32, _L)

        def sig4(j, carry):
            for u in range(4):
                v = rows_v[j * 4 + u]
                sig = 1.0 / (1.0 + jnp.exp(-v))
                rows_v[j * 4 + u] = jnp.where(lane == 0, sig, 1.0)
            return carry

        cp0.wait()
        lax.fori_loop(0, half // 4, sig4, 0)
        cp1.wait()
        lax.fori_loop(half // 4, _GPW // 4, sig4, 0)
        pltpu.sync_copy(rows_v, out_hbm.at[pl.ds(base, _GPW)])

    return ig_gather


_NBS = 4
_ROWS = _NBS * _NUM_INST


def _mil_body(a_ref, b_ref, al_ref, be_ref, wm_ref, x_ref, dcol_ref, drow_ref,
              out_ref):
    ea = jnp.exp(a_ref[0, 0])
    eb = jnp.exp(b_ref[0, 0])

    y = jnp.minimum((-eb) * x_ref[...], 75.0)
    e = jnp.exp(y)
    sw = lax.dot_general(e, wm_ref[...], (((1,), (0,)), ((), ())),
                         preferred_element_type=jnp.float32)
    w = sw[:, 0:1]
    s = sw[:, 1:2]
    z = (w / s).reshape(_NBS, _NUM_INST, 1)

    dzc = (-ea) * dcol_ref[...].reshape(_NBS, _NUM_INST, 1)
    dzr = (-ea) * drow_ref[...]
    gt = (dzr > dzc).astype(jnp.float32)
    gc = jnp.sum(gt, axis=2, keepdims=True)
    sgt = jnp.sum(dzr * gt, axis=2, keepdims=True)
    supp = (gc * dzc + 1.0 > sgt).astype(jnp.float32)
    k = jnp.sum(supp, axis=1, keepdims=True)
    tau = (jnp.sum(supp * dzc, axis=1, keepdims=True) - 1.0) / k
    d = jnp.maximum(dzc - tau, 0.0)

    bag = jnp.sum(d * z, axis=1, keepdims=True)
    ealpha = jnp.exp(al_ref[0, 0])
    beta = be_ref[0, 0]
    out_ref[...] = 1.0 / (1.0 + jnp.exp(-(ealpha * bag + beta)))


def _mil_call(a2, b2, al2, be2, wm, x2, dcol, drow, interpret=False):
    smem = functools.partial(pl.BlockSpec, memory_space=pltpu.SMEM)
    return pl.pallas_call(
        _mil_body,
        grid=(_NUM_BAGS // _NBS,),
        in_specs=[
            smem((1, 1), lambda i: (0, 0)),
            smem((1, 1), lambda i: (0, 0)),
            smem((1, 1), lambda i: (0, 0)),
            smem((1, 1), lambda i: (0, 0)),
            pl.BlockSpec((_G, _L), lambda i: (0, 0)),
            pl.BlockSpec((_ROWS, _G), lambda i: (i, 0)),
            pl.BlockSpec((_ROWS, 1), lambda i: (i, 0)),
            pl.BlockSpec((_NBS, 1, _NUM_INST), lambda i: (i, 0, 0)),
        ],
        out_specs=pl.BlockSpec((_NBS, 1, 1), lambda i: (i, 0, 0)),
        out_shape=jax.ShapeDtypeStruct((_NUM_BAGS, 1, 1), jnp.float32),
        compiler_params=pltpu.CompilerParams(
            dimension_semantics=("parallel",)),
        interpret=interpret,
    )(a2, b2, al2, be2, wm, x2, dcol, drow)


def kernel(distances, gene_expressions, current_genes, a, b, ig_table, alpha,
           beta):
    table16 = jnp.broadcast_to(ig_table[:, None], (_VOCAB, _L))
    wm = _make_ig_gather()(table16, current_genes)
    x2 = gene_expressions.reshape(_NUM_BAGS * _NUM_INST, _G)
    dcol = distances.reshape(_NUM_BAGS * _NUM_INST, 1)
    drow = distances.reshape(_NUM_BAGS, 1, _NUM_INST)
    out = _mil_call(
        a.reshape(1, 1), b.reshape(1, 1), alpha.reshape(1, 1),
        beta.reshape(1, 1), wm, x2, dcol, drow)
    return out.reshape(_NUM_BAGS)


# speedup vs baseline: 1.0406x; 1.0406x over previous
import functools

import jax
import jax.numpy as jnp
from jax import lax
from jax.experimental import pallas as pl
from jax.experimental.pallas import tpu as pltpu
from jax.experimental.pallas import tpu_sc as plsc

_NUM_BAGS = 16
_NUM_INST = 256
_G = 4096
_VOCAB = 128
_NC = 2
_NS = 16
_NW = _NC * _NS
_GPW = _G // _NW
_L = 16


@functools.cache
def _make_ig_gather():
    mesh = plsc.VectorSubcoreMesh(core_axis_name="c", subcore_axis_name="s")
    half = _GPW // 2

    @functools.partial(
        pl.kernel,
        mesh=mesh,
        out_type=jax.ShapeDtypeStruct((_G, _L), jnp.float32),
        scratch_types=[
            pltpu.VMEM((_GPW,), jnp.int32),
            pltpu.VMEM((_GPW, _L), jnp.float32),
            pltpu.SemaphoreType.DMA,
            pltpu.SemaphoreType.DMA,
        ],
        compiler_params=pltpu.CompilerParams(use_tc_tiling_on_sc=False),
    )
    def ig_gather(table_hbm, idx_hbm, out_hbm, idx_v, rows_v, sem0, sem1):
        wid = lax.axis_index("s") * _NC + lax.axis_index("c")
        base = wid * _GPW
        pltpu.sync_copy(idx_hbm.at[pl.ds(base, _GPW)], idx_v)
        cp0 = pltpu.async_copy(
            table_hbm.at[idx_v.at[pl.ds(0, half)]],
            rows_v.at[pl.ds(0, half)], sem0)
        cp1 = pltpu.async_copy(
            table_hbm.at[idx_v.at[pl.ds(half, half)]],
            rows_v.at[pl.ds(half, half)], sem1)
        lane = lax.iota(jnp.int32, _L)

        def sig4(j, carry):
            for u in range(4):
                v = rows_v[j * 4 + u]
                sig = 1.0 / (1.0 + jnp.exp(-v))
                rows_v[j * 4 + u] = jnp.where(lane == 0, sig, 1.0)
            return carry

        cp0.wait()
        lax.fori_loop(0, half // 4, sig4, 0)
        cp1.wait()
        lax.fori_loop(half // 4, _GPW // 4, sig4, 0)
        pltpu.sync_copy(rows_v, out_hbm.at[pl.ds(base, _GPW)])

    return ig_gather


_NBS = 2
_ROWS = _NBS * _NUM_INST


def _mil_body(a_ref, b_ref, al_ref, be_ref, wm_ref, x_ref, dcol_ref, drow_ref,
              out_ref):
    ea = jnp.exp(a_ref[0, 0])
    eb = jnp.exp(b_ref[0, 0])

    y = jnp.minimum((-eb) * x_ref[...], 75.0)
    e = jnp.exp(y)
    sw = lax.dot_general(e, wm_ref[...], (((1,), (0,)), ((), ())),
                         preferred_element_type=jnp.float32)
    w = sw[:, 0:1]
    s = sw[:, 1:2]
    z = (w / s).reshape(_NBS, _NUM_INST, 1)

    dzc = (-ea) * dcol_ref[...].reshape(_NBS, _NUM_INST, 1)
    dzr = (-ea) * drow_ref[...]
    gt = (dzr > dzc).astype(jnp.float32)
    gc = jnp.sum(gt, axis=2, keepdims=True)
    sgt = jnp.sum(dzr * gt, axis=2, keepdims=True)
    supp = (gc * dzc + 1.0 > sgt).astype(jnp.float32)
    k = jnp.sum(supp, axis=1, keepdims=True)
    tau = (jnp.sum(supp * dzc, axis=1, keepdims=True) - 1.0) / k
    d = jnp.maximum(dzc - tau, 0.0)

    bag = jnp.sum(d * z, axis=1, keepdims=True)
    ealpha = jnp.exp(al_ref[0, 0])
    beta = be_ref[0, 0]
    out_ref[...] = 1.0 / (1.0 + jnp.exp(-(ealpha * bag + beta)))


def _mil_call(a2, b2, al2, be2, wm, x2, dcol, drow, interpret=False):
    smem = functools.partial(pl.BlockSpec, memory_space=pltpu.SMEM)
    return pl.pallas_call(
        _mil_body,
        grid=(_NUM_BAGS // _NBS,),
        in_specs=[
            smem((1, 1), lambda i: (0, 0)),
            smem((1, 1), lambda i: (0, 0)),
            smem((1, 1), lambda i: (0, 0)),
            smem((1, 1), lambda i: (0, 0)),
            pl.BlockSpec((_G, _L), lambda i: (0, 0)),
            pl.BlockSpec((_ROWS, _G), lambda i: (i, 0)),
            pl.BlockSpec((_ROWS, 1), lambda i: (i, 0)),
            pl.BlockSpec((_NBS, 1, _NUM_INST), lambda i: (i, 0, 0)),
        ],
        out_specs=pl.BlockSpec((_NBS, 1, 1), lambda i: (i, 0, 0)),
        out_shape=jax.ShapeDtypeStruct((_NUM_BAGS, 1, 1), jnp.float32),
        compiler_params=pltpu.CompilerParams(
            dimension_semantics=("parallel",)),
        interpret=interpret,
    )(a2, b2, al2, be2, wm, x2, dcol, drow)


def kernel(distances, gene_expressions, current_genes, a, b, ig_table, alpha,
           beta):
    table16 = jnp.broadcast_to(ig_table[:, None], (_VOCAB, _L))
    wm = _make_ig_gather()(table16, current_genes)
    x2 = gene_expressions.reshape(_NUM_BAGS * _NUM_INST, _G)
    dcol = distances.reshape(_NUM_BAGS * _NUM_INST, 1)
    drow = distances.reshape(_NUM_BAGS, 1, _NUM_INST)
    out = _mil_call(
        a.reshape(1, 1), b.reshape(1, 1), alpha.reshape(1, 1),
        beta.reshape(1, 1), wm, x2, dcol, drow)
    return out.reshape(_NUM_BAGS)
